# baseline (speedup 1.0000x reference)
import functools

import jax
import jax.numpy as jnp
from jax import lax
from jax.experimental import pallas as pl
from jax.experimental.pallas import tpu as pltpu
from jax.experimental.pallas import tpu_sc as plsc

_VOCAB = 1025
_HID = 768
_NTOK = 16384

_NC = 2
_NS = 16
_NW = _NC * _NS

_B_PER_W = _NTOK // _NW
_CHUNK = 32
_N_CHUNKS = _B_PER_W // _CHUNK
_WPR = _HID // 2
_NBUF_IN = 3
_NBUF_OUT = 2


@functools.cache
def _build():
    mesh = plsc.VectorSubcoreMesh(core_axis_name="c", subcore_axis_name="s")

    @functools.partial(
        pl.kernel,
        mesh=mesh,
        out_type=jax.ShapeDtypeStruct((_NTOK, _HID), jnp.int32),
        scratch_types=(
            [pltpu.VMEM((_B_PER_W,), jnp.int32),
             pltpu.VMEM((_NBUF_IN, _CHUNK, _WPR), jnp.int32),
             pltpu.VMEM((_NBUF_OUT, _CHUNK, _HID), jnp.int32)]
            + [pltpu.SemaphoreType.DMA] * (_NBUF_IN + _NBUF_OUT)
        ),
    )
    def gather_kernel(table_hbm, idx_hbm, out_hbm, idx_v, packed_v, rows_v,
                      *sems):
        wid = lax.axis_index("s") * _NC + lax.axis_index("c")
        base = wid * _B_PER_W
        pltpu.sync_copy(idx_hbm.at[pl.ds(base, _B_PER_W)], idx_v)

        gsems = sems[:_NBUF_IN]
        ssems = sems[_NBUF_IN:]
        gathers = [None] * _NBUF_IN
        scatters = [None] * _NBUF_OUT

        def fire_gather(i):
            b = i % _NBUF_IN
            gathers[b] = pltpu.async_copy(
                table_hbm.at[idx_v.at[pl.ds(i * _CHUNK, _CHUNK)]],
                packed_v.at[b], gsems[b])

        def expand(gb, ob):
            src = packed_v.at[gb]
            dst = rows_v.at[ob]
            hi_mask = jnp.int32(-65536)

            def row_body(r, carry):
                for u in range(_WPR // 16):
                    w = src[r, pl.ds(16 * u, 16)]
                    dst[r, pl.ds(32 * u, 16)] = w << 16
                    dst[r, pl.ds(32 * u + 16, 16)] = w & hi_mask
                return carry

            lax.fori_loop(0, _CHUNK, row_body, 0)

        for j in range(_NBUF_IN - 1):
            fire_gather(j)
        for i in range(_N_CHUNKS):
            gb = i % _NBUF_IN
            ob = i % _NBUF_OUT
            nxt = i + _NBUF_IN - 1
            if nxt < _N_CHUNKS:
                fire_gather(nxt)
            gathers[gb].wait()
            if scatters[ob] is not None:
                scatters[ob].wait()
                scatters[ob] = None
            expand(gb, ob)
            scatters[ob] = pltpu.async_copy(
                rows_v.at[ob], out_hbm.at[pl.ds(base + i * _CHUNK, _CHUNK)],
                ssems[ob])
        for s in scatters:
            if s is not None:
                s.wait()

    return gather_kernel


def _pack_table(embed_tokens):
    t = embed_tokens.astype(jnp.bfloat16)
    t = t.reshape(_VOCAB, _HID // 32, 2, 16).transpose(0, 1, 3, 2)
    return jax.lax.bitcast_convert_type(t, jnp.int32).reshape(_VOCAB, _WPR)


def kernel(input_ids, positions, embed_tokens):
    del positions
    raw = _build()(_pack_table(embed_tokens), input_ids.astype(jnp.int32))
    return jax.lax.bitcast_convert_type(raw, jnp.float32)


# speedup vs baseline: 2.0152x; 2.0152x over previous
import functools

import jax
import jax.numpy as jnp
from jax import lax
from jax.experimental import pallas as pl
from jax.experimental.pallas import tpu as pltpu
from jax.experimental.pallas import tpu_sc as plsc

_VOCAB = 1025
_HID = 768
_NTOK = 16384

_NC = 2
_NS = 16
_NW = _NC * _NS

_B_PER_W = _NTOK // _NW
_CHUNK = 64
_N_CHUNKS = _B_PER_W // _CHUNK


@functools.cache
def _build():
    mesh = plsc.VectorSubcoreMesh(core_axis_name="c", subcore_axis_name="s")

    @functools.partial(
        pl.kernel,
        mesh=mesh,
        out_type=jax.ShapeDtypeStruct((_NTOK, _HID), jnp.float32),
        scratch_types=[
            pltpu.VMEM((_B_PER_W,), jnp.int32),
            pltpu.VMEM((2, _CHUNK, _HID), jnp.float32),
            pltpu.SemaphoreType.DMA,
            pltpu.SemaphoreType.DMA,
            pltpu.SemaphoreType.DMA,
            pltpu.SemaphoreType.DMA,
        ],
    )
    def gather_kernel(table_hbm, idx_hbm, out_hbm, idx_v, rows_v,
                      gsem0, gsem1, ssem0, ssem1):
        wid = lax.axis_index("s") * _NC + lax.axis_index("c")
        base = wid * _B_PER_W
        pltpu.sync_copy(idx_hbm.at[pl.ds(base, _B_PER_W)], idx_v)

        gsems = [gsem0, gsem1]
        ssems = [ssem0, ssem1]
        gathers = [None, None]
        scatters = [None, None]

        gathers[0] = pltpu.async_copy(
            table_hbm.at[idx_v.at[pl.ds(0, _CHUNK)]], rows_v.at[0], gsems[0])
        for i in range(_N_CHUNKS):
            buf = i % 2
            nbuf = (i + 1) % 2
            if i + 1 < _N_CHUNKS:
                if scatters[nbuf] is not None:
                    scatters[nbuf].wait()
                    scatters[nbuf] = None
                gathers[nbuf] = pltpu.async_copy(
                    table_hbm.at[idx_v.at[pl.ds((i + 1) * _CHUNK, _CHUNK)]],
                    rows_v.at[nbuf], gsems[nbuf])
            gathers[buf].wait()
            scatters[buf] = pltpu.async_copy(
                rows_v.at[buf], out_hbm.at[pl.ds(base + i * _CHUNK, _CHUNK)],
                ssems[buf])
        for s in scatters:
            if s is not None:
                s.wait()

    return gather_kernel


def kernel(input_ids, positions, embed_tokens):
    del positions
    return _build()(embed_tokens, input_ids.astype(jnp.int32))


# speedup vs baseline: 2.0224x; 1.0036x over previous
import functools

import jax
import jax.numpy as jnp
from jax import lax
from jax.experimental import pallas as pl
from jax.experimental.pallas import tpu as pltpu
from jax.experimental.pallas import tpu_sc as plsc

_VOCAB = 1025
_HID = 768
_NTOK = 16384

_NC = 2
_NS = 16
_NW = _NC * _NS

_B_PER_W = _NTOK // _NW
_CHUNK = 32
_DEPTH = 4
_N_CHUNKS = _B_PER_W // _CHUNK


@functools.cache
def _build():
    mesh = plsc.VectorSubcoreMesh(core_axis_name="c", subcore_axis_name="s")

    @functools.partial(
        pl.kernel,
        mesh=mesh,
        out_type=jax.ShapeDtypeStruct((_NTOK, _HID), jnp.float32),
        scratch_types=[
            pltpu.VMEM((_B_PER_W,), jnp.int32),
            pltpu.VMEM((_DEPTH, _CHUNK, _HID), jnp.float32),
        ] + [pltpu.SemaphoreType.DMA] * (2 * _DEPTH),
    )
    def gather_kernel(table_hbm, idx_hbm, out_hbm, idx_v, rows_v, *sems):
        wid = lax.axis_index("s") * _NC + lax.axis_index("c")
        base = wid * _B_PER_W
        pltpu.sync_copy(idx_hbm.at[pl.ds(base, _B_PER_W)], idx_v)

        gsems = sems[:_DEPTH]
        ssems = sems[_DEPTH:]
        gathers = [None] * _DEPTH
        scatters = [None] * _DEPTH

        for i in range(_DEPTH - 1):
            gathers[i] = pltpu.async_copy(
                table_hbm.at[idx_v.at[pl.ds(i * _CHUNK, _CHUNK)]],
                rows_v.at[i], gsems[i])
        for i in range(_N_CHUNKS):
            buf = i % _DEPTH
            nxt = i + _DEPTH - 1
            if nxt < _N_CHUNKS:
                nbuf = nxt % _DEPTH
                if scatters[nbuf] is not None:
                    scatters[nbuf].wait()
                    scatters[nbuf] = None
                gathers[nbuf] = pltpu.async_copy(
                    table_hbm.at[idx_v.at[pl.ds(nxt * _CHUNK, _CHUNK)]],
                    rows_v.at[nbuf], gsems[nbuf])
            gathers[buf].wait()
            scatters[buf] = pltpu.async_copy(
                rows_v.at[buf], out_hbm.at[pl.ds(base + i * _CHUNK, _CHUNK)],
                ssems[buf])
        for s in scatters:
            if s is not None:
                s.wait()

    return gather_kernel


def kernel(input_ids, positions, embed_tokens):
    del positions
    return _build()(embed_tokens, input_ids.astype(jnp.int32))
